# baseline (speedup 1.0000x reference)
import jax
import jax.numpy as jnp
from jax.experimental import pallas as pl
from jax.experimental.pallas import tpu as pltpu
from jax.experimental.pallas import tpu_sc as plsc

_K1 = 17
_K = 16
_TQ = 1024
_TN = 512
_TW = 512
_GW = 128

_HI = jax.lax.Precision.HIGHEST


def _knn_body(xyz_ref, xyzT_ref, idx_ref, w_ref):
    xz = xyz_ref[0]
    qT = xyzT_ref[0]
    n = xz.shape[1]
    tq = qT.shape[0]
    sq = jnp.sum(xz * xz, axis=0, keepdims=True)
    sq_q = jnp.sum(qT * qT, axis=1, keepdims=True)
    acc = qT[:, 0:1] * xz[0:1, :]
    acc = acc + qT[:, 1:2] * xz[1:2, :]
    acc = acc + qT[:, 2:3] * xz[2:3, :]
    d2 = sq_q + sq - 2.0 * acc
    d = jnp.sqrt(jnp.maximum(d2, 0.0))
    iota = jax.lax.broadcasted_iota(jnp.int32, (1, n), 1)
    vals, inds = [], []
    for _ in range(_K1):
        v = jnp.min(d, axis=1, keepdims=True)
        i = jnp.argmin(d, axis=1).astype(jnp.int32)[:, None]
        vals.append(v)
        inds.append(i)
        d = jnp.where(iota == i, jnp.inf, d)
    dv = jnp.concatenate(vals[1:], axis=1)
    iv = jnp.concatenate(inds[1:], axis=1)
    dn = 1.0 / dv
    wsum = jnp.sum(dn, axis=1, keepdims=True)
    w_ref[0] = dn / (wsum + 1e-7)
    idx_ref[0] = iv


def _knn(xyz, xyzT):
    b, _, n = xyz.shape
    return pl.pallas_call(
        _knn_body,
        grid=(b, n // _TQ),
        in_specs=[
            pl.BlockSpec((1, 3, n), lambda i, j: (i, 0, 0)),
            pl.BlockSpec((1, _TQ, 3), lambda i, j: (i, j, 0)),
        ],
        out_specs=[
            pl.BlockSpec((1, _TQ, _K), lambda i, j: (i, j, 0)),
            pl.BlockSpec((1, _TQ, _K), lambda i, j: (i, j, 0)),
        ],
        out_shape=[
            jax.ShapeDtypeStruct((b, n, _K), jnp.int32),
            jax.ShapeDtypeStruct((b, n, _K), jnp.float32),
        ],
        compiler_params=pltpu.CompilerParams(
            dimension_semantics=("parallel", "arbitrary")),
    )(xyz, xyzT)


def _sc_group_max(featP, idx_flat, w_flat):
    num = idx_flat.shape[1]
    nq = num // _K
    qw = _GW // _K
    mesh = plsc.VectorSubcoreMesh(core_axis_name="core",
                                  subcore_axis_name="subcore")

    @pl.kernel(out_type=jax.ShapeDtypeStruct((nq, 64), jnp.float32),
               mesh=mesh,
               scratch_types=[pltpu.VMEM((_GW, 128), jnp.float32)])
    def kern(x_hbm, i_hbm, w_hbm, o_hbm, g_vmem):
        def body(i_vmem, w_vmem, o_vmem):
            pltpu.sync_copy(x_hbm.at[i_vmem.at[0]], g_vmem)
            for q in range(qw):
                wv = w_vmem[0, pl.ds(q * _K, _K)]
                for c in range(0, 64, 16):
                    acc = wv[0] * g_vmem[q * _K, c:c + 16]
                    for k in range(1, _K):
                        acc = jnp.maximum(
                            acc, wv[k] * g_vmem[q * _K + k, c:c + 16])
                    o_vmem[q, c:c + 16] = acc

        pltpu.emit_pipeline(
            body,
            grid=(num // _GW,),
            in_specs=[pl.BlockSpec((1, _GW), lambda i: (0, i)),
                      pl.BlockSpec((1, _GW), lambda i: (0, i))],
            out_specs=[pl.BlockSpec((qw, 64), lambda i: (i, 0))],
            core_axis_name=("core", "subcore"),
            dimension_semantics=(pltpu.PARALLEL,),
        )(i_hbm, w_hbm, o_hbm)

    return kern(featP, idx_flat, w_flat)


def _attn_body(fea2_ref, fw_ref, fb_ref, gw_ref, gb_ref, hw_ref, hb_ref,
               gamma_ref, aw_ref, ab_ref, bw_ref, bb_ref, out_ref, oacc_ref):
    t = pl.program_id(1)
    nt = pl.num_programs(1)
    relu = jax.nn.relu
    fea2 = fea2_ref[0]
    fea2_t = fea2_ref[0, :, pl.ds(t * _TN, _TN)]
    g = relu(jnp.dot(gw_ref[...], fea2_t, precision=_HI) + gb_ref[...])
    h = relu(jnp.dot(hw_ref[...], fea2_t, precision=_HI) + hb_ref[...])
    f = relu(jnp.dot(fw_ref[...], fea2, precision=_HI) + fb_ref[...])
    s = jax.lax.dot_general(g, f, (((0,), (0,)), ((), ())))
    mx = jnp.max(s, axis=1, keepdims=True)
    e = jnp.exp(s - mx)
    z = jnp.sum(e, axis=1, keepdims=True)
    beta = e * (1.0 / z)
    contrib = jnp.dot(h, beta)

    @pl.when(t == 0)
    def _():
        oacc_ref[...] = contrib

    @pl.when(t != 0)
    def _():
        oacc_ref[...] = oacc_ref[...] + contrib

    @pl.when(t == nt - 1)
    def _():
        x = gamma_ref[0, 0] * oacc_ref[...] + fea2
        y = relu(jnp.dot(aw_ref[...], x, precision=_HI) + ab_ref[...])
        out_ref[0] = jnp.dot(bw_ref[...], y, precision=_HI) + bb_ref[...]


def _attn(fea2, fw, fb, gw, gb, hw, hb, gamma, aw, ab, bw, bb):
    b, c, n = fea2.shape
    full = lambda shape: pl.BlockSpec(shape, lambda i, j: tuple(0 for _ in shape))
    return pl.pallas_call(
        _attn_body,
        grid=(b, n // _TN),
        in_specs=[
            pl.BlockSpec((1, c, n), lambda i, j: (i, 0, 0)),
            full(fw.shape), full(fb.shape), full(gw.shape), full(gb.shape),
            full(hw.shape), full(hb.shape), full(gamma.shape),
            full(aw.shape), full(ab.shape), full(bw.shape), full(bb.shape),
        ],
        out_specs=pl.BlockSpec((1, 3, n), lambda i, j: (i, 0, 0)),
        out_shape=jax.ShapeDtypeStruct((b, 3, n), jnp.float32),
        scratch_shapes=[pltpu.VMEM((c, n), jnp.float32)],
        compiler_params=pltpu.CompilerParams(
            dimension_semantics=("parallel", "arbitrary")),
    )(fea2, fw, fb, gw, gb, hw, hb, gamma, aw, ab, bw, bb)


def kernel(feature, xyz, convF_w, convF_b, convG_w, convG_b, convH_w,
           convH_b, gamma, conv2a_w, conv2a_b, conv2b_w, conv2b_b):
    b, c, n = feature.shape
    xyzT = jnp.transpose(xyz, (0, 2, 1))
    featT = jnp.transpose(feature, (0, 2, 1)).reshape(b * n, c)
    featP = jnp.pad(featT, ((0, 0), (0, 128 - c)))
    outs = []
    for bi in range(b):
        idx, w = _knn(xyz[bi:bi + 1], xyzT[bi:bi + 1])
        idx_flat = (idx + bi * n).reshape(1, n * _K)
        w_flat = w.reshape(1, n * _K)
        fea2T = _sc_group_max(featP, idx_flat, w_flat)
        fea2 = jnp.transpose(fea2T, (1, 0))[None]
        outs.append(_attn(
            fea2, convF_w, convF_b.reshape(-1, 1), convG_w,
            convG_b.reshape(-1, 1), convH_w, convH_b.reshape(-1, 1),
            gamma.reshape(1, 1), conv2a_w, conv2a_b.reshape(-1, 1),
            conv2b_w, conv2b_b.reshape(-1, 1)))
    return jnp.concatenate(outs, axis=0)


# speedup vs baseline: 1.0064x; 1.0064x over previous
import jax
import jax.numpy as jnp
from jax.experimental import pallas as pl
from jax.experimental.pallas import tpu as pltpu
from jax.experimental.pallas import tpu_sc as plsc

_K1 = 17
_K = 16
_TQ = 1024
_TN = 1024
_TW = 512
_GW = 128

_HI = jax.lax.Precision.HIGHEST


def _knn_body(xyz_ref, xyzT_ref, idx_ref, w_ref):
    xz = xyz_ref[0]
    qT = xyzT_ref[0]
    n = xz.shape[1]
    tq = qT.shape[0]
    sq = jnp.sum(xz * xz, axis=0, keepdims=True)
    sq_q = jnp.sum(qT * qT, axis=1, keepdims=True)
    acc = qT[:, 0:1] * xz[0:1, :]
    acc = acc + qT[:, 1:2] * xz[1:2, :]
    acc = acc + qT[:, 2:3] * xz[2:3, :]
    d2 = sq_q + sq - 2.0 * acc
    d = jnp.sqrt(jnp.maximum(d2, 0.0))
    iota = jax.lax.broadcasted_iota(jnp.int32, (1, n), 1)
    vals, inds = [], []
    for _ in range(_K1):
        v = jnp.min(d, axis=1, keepdims=True)
        i = jnp.argmin(d, axis=1).astype(jnp.int32)[:, None]
        vals.append(v)
        inds.append(i)
        d = jnp.where(iota == i, jnp.inf, d)
    dv = jnp.concatenate(vals[1:], axis=1)
    iv = jnp.concatenate(inds[1:], axis=1)
    dn = 1.0 / dv
    wsum = jnp.sum(dn, axis=1, keepdims=True)
    w_ref[0] = dn / (wsum + 1e-7)
    idx_ref[0] = iv


def _knn(xyz, xyzT):
    b, _, n = xyz.shape
    return pl.pallas_call(
        _knn_body,
        grid=(b, n // _TQ),
        in_specs=[
            pl.BlockSpec((1, 3, n), lambda i, j: (i, 0, 0)),
            pl.BlockSpec((1, _TQ, 3), lambda i, j: (i, j, 0)),
        ],
        out_specs=[
            pl.BlockSpec((1, _TQ, _K), lambda i, j: (i, j, 0)),
            pl.BlockSpec((1, _TQ, _K), lambda i, j: (i, j, 0)),
        ],
        out_shape=[
            jax.ShapeDtypeStruct((b, n, _K), jnp.int32),
            jax.ShapeDtypeStruct((b, n, _K), jnp.float32),
        ],
        compiler_params=pltpu.CompilerParams(
            dimension_semantics=("parallel", "arbitrary")),
    )(xyz, xyzT)


def _sc_group_max(featP, idx_flat, w_flat):
    num = idx_flat.shape[1]
    nq = num // _K
    qw = _GW // _K
    mesh = plsc.VectorSubcoreMesh(core_axis_name="core",
                                  subcore_axis_name="subcore")

    @pl.kernel(out_type=jax.ShapeDtypeStruct((nq, 64), jnp.float32),
               mesh=mesh,
               scratch_types=[pltpu.VMEM((_GW, 128), jnp.float32)])
    def kern(x_hbm, i_hbm, w_hbm, o_hbm, g_vmem):
        def body(i_vmem, w_vmem, o_vmem):
            pltpu.sync_copy(x_hbm.at[i_vmem.at[0]], g_vmem)
            for q in range(qw):
                wv = w_vmem[0, pl.ds(q * _K, _K)]
                for c in range(0, 64, 16):
                    acc = wv[0] * g_vmem[q * _K, c:c + 16]
                    for k in range(1, _K):
                        acc = jnp.maximum(
                            acc, wv[k] * g_vmem[q * _K + k, c:c + 16])
                    o_vmem[q, c:c + 16] = acc

        pltpu.emit_pipeline(
            body,
            grid=(num // _GW,),
            in_specs=[pl.BlockSpec((1, _GW), lambda i: (0, i)),
                      pl.BlockSpec((1, _GW), lambda i: (0, i))],
            out_specs=[pl.BlockSpec((qw, 64), lambda i: (i, 0))],
            core_axis_name=("core", "subcore"),
            dimension_semantics=(pltpu.PARALLEL,),
        )(i_hbm, w_hbm, o_hbm)

    return kern(featP, idx_flat, w_flat)


def _attn_body(fea2_ref, fw_ref, fb_ref, gw_ref, gb_ref, hw_ref, hb_ref,
               gamma_ref, aw_ref, ab_ref, bw_ref, bb_ref, out_ref, oacc_ref):
    t = pl.program_id(1)
    nt = pl.num_programs(1)
    relu = jax.nn.relu
    fea2 = fea2_ref[0]
    fea2_t = fea2_ref[0, :, pl.ds(t * _TN, _TN)]
    g = relu(jnp.dot(gw_ref[...], fea2_t, precision=_HI) + gb_ref[...])
    h = relu(jnp.dot(hw_ref[...], fea2_t, precision=_HI) + hb_ref[...])
    f = relu(jnp.dot(fw_ref[...], fea2, precision=_HI) + fb_ref[...])
    s = jax.lax.dot_general(g, f, (((0,), (0,)), ((), ())))
    mx = jnp.max(s, axis=1, keepdims=True)
    e = jnp.exp(s - mx)
    z = jnp.sum(e, axis=1, keepdims=True)
    beta = e * (1.0 / z)
    contrib = jnp.dot(h, beta)

    @pl.when(t == 0)
    def _():
        oacc_ref[...] = contrib

    @pl.when(t != 0)
    def _():
        oacc_ref[...] = oacc_ref[...] + contrib

    @pl.when(t == nt - 1)
    def _():
        x = gamma_ref[0, 0] * oacc_ref[...] + fea2
        y = relu(jnp.dot(aw_ref[...], x, precision=_HI) + ab_ref[...])
        out_ref[0] = jnp.dot(bw_ref[...], y, precision=_HI) + bb_ref[...]


def _attn(fea2, fw, fb, gw, gb, hw, hb, gamma, aw, ab, bw, bb):
    b, c, n = fea2.shape
    full = lambda shape: pl.BlockSpec(shape, lambda i, j: tuple(0 for _ in shape))
    return pl.pallas_call(
        _attn_body,
        grid=(b, n // _TN),
        in_specs=[
            pl.BlockSpec((1, c, n), lambda i, j: (i, 0, 0)),
            full(fw.shape), full(fb.shape), full(gw.shape), full(gb.shape),
            full(hw.shape), full(hb.shape), full(gamma.shape),
            full(aw.shape), full(ab.shape), full(bw.shape), full(bb.shape),
        ],
        out_specs=pl.BlockSpec((1, 3, n), lambda i, j: (i, 0, 0)),
        out_shape=jax.ShapeDtypeStruct((b, 3, n), jnp.float32),
        scratch_shapes=[pltpu.VMEM((c, n), jnp.float32)],
        compiler_params=pltpu.CompilerParams(
            dimension_semantics=("parallel", "arbitrary")),
    )(fea2, fw, fb, gw, gb, hw, hb, gamma, aw, ab, bw, bb)


def kernel(feature, xyz, convF_w, convF_b, convG_w, convG_b, convH_w,
           convH_b, gamma, conv2a_w, conv2a_b, conv2b_w, conv2b_b):
    b, c, n = feature.shape
    xyzT = jnp.transpose(xyz, (0, 2, 1))
    featT = jnp.transpose(feature, (0, 2, 1)).reshape(b * n, c)
    featP = jnp.pad(featT, ((0, 0), (0, 128 - c)))
    outs = []
    for bi in range(b):
        idx, w = _knn(xyz[bi:bi + 1], xyzT[bi:bi + 1])
        idx_flat = (idx + bi * n).reshape(1, n * _K)
        w_flat = w.reshape(1, n * _K)
        fea2T = _sc_group_max(featP, idx_flat, w_flat)
        fea2 = jnp.transpose(fea2T, (1, 0))[None]
        outs.append(_attn(
            fea2, convF_w, convF_b.reshape(-1, 1), convG_w,
            convG_b.reshape(-1, 1), convH_w, convH_b.reshape(-1, 1),
            gamma.reshape(1, 1), conv2a_w, conv2a_b.reshape(-1, 1),
            conv2b_w, conv2b_b.reshape(-1, 1)))
    return jnp.concatenate(outs, axis=0)


# speedup vs baseline: 1.0071x; 1.0007x over previous
import jax
import jax.numpy as jnp
from jax.experimental import pallas as pl
from jax.experimental.pallas import tpu as pltpu
from jax.experimental.pallas import tpu_sc as plsc

_K1 = 17
_K = 16
_TQ = 1024
_TN = 1024
_GW = 128

_HI = jax.lax.Precision.HIGHEST


def _knn_body(xyz_ref, xyzT_ref, idx_ref, w_ref):
    xz = xyz_ref[0]
    qT = xyzT_ref[0]
    n = xz.shape[1]
    tq = qT.shape[0]
    sq = jnp.sum(xz * xz, axis=0, keepdims=True)
    sq_q = jnp.sum(qT * qT, axis=1, keepdims=True)
    acc = qT[:, 0:1] * xz[0:1, :]
    acc = acc + qT[:, 1:2] * xz[1:2, :]
    acc = acc + qT[:, 2:3] * xz[2:3, :]
    d2 = sq_q + sq - 2.0 * acc
    d = jnp.sqrt(jnp.maximum(d2, 0.0))
    iota = jax.lax.broadcasted_iota(jnp.int32, (1, n), 1)
    vals, inds = [], []
    for _ in range(_K1):
        v = jnp.min(d, axis=1, keepdims=True)
        i = jnp.argmin(d, axis=1).astype(jnp.int32)[:, None]
        vals.append(v)
        inds.append(i)
        d = jnp.where(iota == i, jnp.inf, d)
    dv = jnp.concatenate(vals[1:], axis=1)
    iv = jnp.concatenate(inds[1:], axis=1)
    dn = 1.0 / dv
    wsum = jnp.sum(dn, axis=1, keepdims=True)
    w_ref[0] = dn / (wsum + 1e-7)
    idx_ref[0] = iv


def _knn(xyz, xyzT):
    b, _, n = xyz.shape
    return pl.pallas_call(
        _knn_body,
        grid=(b, n // _TQ),
        in_specs=[
            pl.BlockSpec((1, 3, n), lambda i, j: (i, 0, 0)),
            pl.BlockSpec((1, _TQ, 3), lambda i, j: (i, j, 0)),
        ],
        out_specs=[
            pl.BlockSpec((1, _TQ, _K), lambda i, j: (i, j, 0)),
            pl.BlockSpec((1, _TQ, _K), lambda i, j: (i, j, 0)),
        ],
        out_shape=[
            jax.ShapeDtypeStruct((b, n, _K), jnp.int32),
            jax.ShapeDtypeStruct((b, n, _K), jnp.float32),
        ],
        compiler_params=pltpu.CompilerParams(
            dimension_semantics=("parallel", "arbitrary")),
    )(xyz, xyzT)


def _sc_group_max(featP, idx_flat, w_flat):
    num = idx_flat.shape[1]
    nq = num // _K
    qw = _GW // _K
    mesh = plsc.VectorSubcoreMesh(core_axis_name="core",
                                  subcore_axis_name="subcore")

    @pl.kernel(out_type=jax.ShapeDtypeStruct((nq, 64), jnp.float32),
               mesh=mesh,
               scratch_types=[pltpu.VMEM((_GW, 128), jnp.float32)])
    def kern(x_hbm, i_hbm, w_hbm, o_hbm, g_vmem):
        def body(i_vmem, w_vmem, o_vmem):
            pltpu.sync_copy(x_hbm.at[i_vmem.at[0]], g_vmem)
            for q in range(qw):
                wv = w_vmem[0, pl.ds(q * _K, _K)]
                for c in range(0, 64, 16):
                    acc = wv[0] * g_vmem[q * _K, c:c + 16]
                    for k in range(1, _K):
                        acc = jnp.maximum(
                            acc, wv[k] * g_vmem[q * _K + k, c:c + 16])
                    o_vmem[q, c:c + 16] = acc

        pltpu.emit_pipeline(
            body,
            grid=(num // _GW,),
            in_specs=[pl.BlockSpec((1, _GW), lambda i: (0, i)),
                      pl.BlockSpec((1, _GW), lambda i: (0, i))],
            out_specs=[pl.BlockSpec((qw, 64), lambda i: (i, 0))],
            core_axis_name=("core", "subcore"),
            dimension_semantics=(pltpu.PARALLEL,),
        )(i_hbm, w_hbm, o_hbm)

    return kern(featP, idx_flat, w_flat)


def _attn_body(fea2_ref, fw_ref, fb_ref, gw_ref, gb_ref, hw_ref, hb_ref,
               gamma_ref, aw_ref, ab_ref, bw_ref, bb_ref, out_ref, oacc_ref):
    t = pl.program_id(1)
    nt = pl.num_programs(1)
    relu = jax.nn.relu
    fea2 = fea2_ref[0]
    fea2_t = fea2_ref[0, :, pl.ds(t * _TN, _TN)]
    g = relu(jnp.dot(gw_ref[...], fea2_t, precision=_HI) + gb_ref[...])
    h = relu(jnp.dot(hw_ref[...], fea2_t, precision=_HI) + hb_ref[...])
    f = relu(jnp.dot(fw_ref[...], fea2, precision=_HI) + fb_ref[...])
    s = jax.lax.dot_general(g, f, (((0,), (0,)), ((), ())))
    mx = jnp.max(s, axis=1, keepdims=True)
    e = jnp.exp(s - mx)
    z = jnp.sum(e, axis=1, keepdims=True)
    beta = e * (1.0 / z)
    contrib = jnp.dot(h, beta)

    @pl.when(t == 0)
    def _():
        oacc_ref[...] = contrib

    @pl.when(t != 0)
    def _():
        oacc_ref[...] = oacc_ref[...] + contrib

    @pl.when(t == nt - 1)
    def _():
        x = gamma_ref[0, 0] * oacc_ref[...] + fea2
        y = relu(jnp.dot(aw_ref[...], x, precision=_HI) + ab_ref[...])
        out_ref[0] = jnp.dot(bw_ref[...], y, precision=_HI) + bb_ref[...]


def _attn(fea2, fw, fb, gw, gb, hw, hb, gamma, aw, ab, bw, bb):
    b, c, n = fea2.shape
    full = lambda shape: pl.BlockSpec(shape, lambda i, j: tuple(0 for _ in shape))
    return pl.pallas_call(
        _attn_body,
        grid=(b, n // _TN),
        in_specs=[
            pl.BlockSpec((1, c, n), lambda i, j: (i, 0, 0)),
            full(fw.shape), full(fb.shape), full(gw.shape), full(gb.shape),
            full(hw.shape), full(hb.shape), full(gamma.shape),
            full(aw.shape), full(ab.shape), full(bw.shape), full(bb.shape),
        ],
        out_specs=pl.BlockSpec((1, 3, n), lambda i, j: (i, 0, 0)),
        out_shape=jax.ShapeDtypeStruct((b, 3, n), jnp.float32),
        scratch_shapes=[pltpu.VMEM((c, n), jnp.float32)],
        compiler_params=pltpu.CompilerParams(
            dimension_semantics=("parallel", "arbitrary")),
    )(fea2, fw, fb, gw, gb, hw, hb, gamma, aw, ab, bw, bb)


def kernel(feature, xyz, convF_w, convF_b, convG_w, convG_b, convH_w,
           convH_b, gamma, conv2a_w, conv2a_b, conv2b_w, conv2b_b):
    b, c, n = feature.shape
    xyzT = jnp.transpose(xyz, (0, 2, 1))
    featT = jnp.transpose(feature, (0, 2, 1)).reshape(b * n, c)
    featP = jnp.pad(featT, ((0, 0), (0, 128 - c)))
    outs = []
    for bi in range(b):
        idx, w = _knn(xyz[bi:bi + 1], xyzT[bi:bi + 1])
        idx_flat = (idx + bi * n).reshape(1, n * _K)
        w_flat = w.reshape(1, n * _K)
        fea2T = _sc_group_max(featP, idx_flat, w_flat)
        fea2 = jnp.transpose(fea2T, (1, 0))[None]
        outs.append(_attn(
            fea2, convF_w, convF_b.reshape(-1, 1), convG_w,
            convG_b.reshape(-1, 1), convH_w, convH_b.reshape(-1, 1),
            gamma.reshape(1, 1), conv2a_w, conv2a_b.reshape(-1, 1),
            conv2b_w, conv2b_b.reshape(-1, 1)))
    return jnp.concatenate(outs, axis=0)
